# Initial kernel scaffold; baseline (speedup 1.0000x reference)
import jax
import jax.numpy as jnp
from jax.experimental import pallas as pl


def kernel(user_table, item_table, W_vl, Wu0a, Wu0b, Wu1a, Wu1b, Wi0a, Wi0b, Wi1a, Wi1b, Wuc, buc, Wic, bic, adj_src, adj_dst, uu0_src, uu0_dst, uu1_src, uu1_dst, ii0_src, ii0_dst, ii1_src, ii1_dst, user, positive, negative):
    raise NotImplementedError("write your pallas kernel here")


# speedup vs baseline: 1.3214x; 1.3214x over previous
import functools

import jax
import jax.numpy as jnp
from jax import lax
from jax.experimental import pallas as pl
from jax.experimental.pallas import tpu as pltpu
from jax.experimental.pallas import tpu_sc as plsc

NU = 10000
NI = 10000
N = NU + NI
D = 128
H = 64
B = 4096
E = 320000
EH = 160000
IB = 64
IB2 = 32
GCN_LAYERS = 3
TEMP = 0.2
REG = 1e-4
SSL = 0.1
IBL = 0.01
INTRA = 0.1


def _info_nce(z1, z2, t):
    z1 = z1 / (jnp.linalg.norm(z1, axis=1, keepdims=True) + 1e-8)
    z2 = z2 / (jnp.linalg.norm(z2, axis=1, keepdims=True) + 1e-8)
    pos = jnp.sum(z1 * z2, axis=1) / t
    ttl = (z1 @ z2.T) / t
    return jnp.mean(jax.scipy.special.logsumexp(ttl, axis=1) - pos)


def _stack5_kernel(x_ref, o_ref):
    o_ref[...] = x_ref[...]


def _stack5(vals):
    x = jnp.broadcast_to(jnp.pad(jnp.stack(vals), (0, 3))[:, None], (8, 128))
    out = pl.pallas_call(
        _stack5_kernel,
        out_shape=jax.ShapeDtypeStruct((8, 128), jnp.float32),
    )(x)
    return out[:5, 0]


def kernel(user_table, item_table, W_vl, Wu0a, Wu0b, Wu1a, Wu1b, Wi0a, Wi0b,
           Wi1a, Wi1b, Wuc, buc, Wic, bic, adj_src, adj_dst, uu0_src, uu0_dst,
           uu1_src, uu1_dst, ii0_src, ii0_dst, ii1_src, ii1_dst, user,
           positive, negative):
    n = N
    ones = jnp.ones(adj_src.shape, jnp.float32)
    deg_s = jax.ops.segment_sum(ones, adj_src, num_segments=n)
    deg_d = jax.ops.segment_sum(ones, adj_dst, num_segments=n)
    rs = lax.rsqrt(jnp.maximum(deg_s, 1.0))
    rd = lax.rsqrt(jnp.maximum(deg_d, 1.0))

    x = jnp.concatenate([user_table, item_table], axis=0)
    S = jnp.zeros((n, D), jnp.float32)
    for _ in range(GCN_LAYERS):
        xs = x * rs[:, None]
        acc = jax.ops.segment_sum(xs[adj_src], adj_dst, num_segments=n)
        x = acc * rd[:, None]
        S = S + x

    def view_learner(src, dst, emb):
        h = emb @ W_vl
        return jax.nn.sigmoid(jnp.sum(h[src] * h[dst], axis=1))

    def autoenc(src, dst, ew, emb, W1, W2, nn_):
        h = jax.ops.segment_sum(emb[src] * ew[:, None], dst, num_segments=nn_)
        h = jax.nn.relu(h @ W1)
        h2 = jax.ops.segment_sum(h[src] * ew[:, None], dst, num_segments=nn_)
        return h2 @ W2

    hu0 = autoenc(uu0_src, uu0_dst,
                  view_learner(uu0_src, uu0_dst, user_table),
                  user_table, Wu0a, Wu0b, NU)
    hu1 = autoenc(uu1_src, uu1_dst,
                  view_learner(uu1_src, uu1_dst, user_table),
                  user_table, Wu1a, Wu1b, NU)
    hi0 = autoenc(ii0_src, ii0_dst,
                  view_learner(ii0_src, ii0_dst, item_table),
                  item_table, Wi0a, Wi0b, NI)
    hi1 = autoenc(ii1_src, ii1_dst,
                  view_learner(ii1_src, ii1_dst, item_table),
                  item_table, Wi1a, Wi1b, NI)

    user_node = 0.5 * (hu0 + hu1) @ Wuc + buc
    item_node = 0.5 * (hi0 + hi1) @ Wic + bic
    user_mu = user_node[:, :IB2]
    user_std = jax.nn.softplus(user_node[:, IB2:] - IB2)
    item_mu = item_node[:, :IB2]
    item_std = jax.nn.softplus(item_node[:, IB2:] - IB2)
    ukl = -0.5 * jnp.mean(jnp.sum(
        1.0 + 2.0 * jnp.log(user_std) - user_mu ** 2 - user_std ** 2,
        axis=1)) / jnp.log(2.0)
    ikl = -0.5 * jnp.mean(jnp.sum(
        1.0 + 2.0 * jnp.log(item_std) - item_mu ** 2 - item_std ** 2,
        axis=1)) / jnp.log(2.0)
    ib_loss = IBL * (ukl + ikl)

    final = S / 3.0
    ue = final[user]
    pe = final[NU + positive]
    ne = final[NU + negative]
    g_hu0 = hu0[user]
    g_hu1 = hu1[user]
    g_hi0 = hi0[positive]
    g_hi1 = hi1[positive]
    ego_u = user_table[user]
    ego_p = item_table[positive]
    ego_n = item_table[negative]

    intra_loss = INTRA * (_info_nce(g_hu0, g_hu1, TEMP)
                          + _info_nce(g_hi0, g_hi1, TEMP))
    bpr_loss = -jnp.mean(jax.nn.log_sigmoid(
        jnp.sum(ue * pe, axis=1) - jnp.sum(ue * ne, axis=1)))
    reg_loss = REG * 0.5 * (jnp.sum(ego_u ** 2) + jnp.sum(ego_p ** 2)
                            + jnp.sum(ego_n ** 2)) / B
    ssl_loss = SSL * (_info_nce(ue + g_hu0, ue + g_hu1, TEMP)
                      + _info_nce(pe + g_hi0, pe + g_hi1, TEMP))
    return _stack5([bpr_loss, reg_loss, ssl_loss, intra_loss, ib_loss])


# speedup vs baseline: 1.6002x; 1.2110x over previous
import functools

import jax
import jax.numpy as jnp
from jax import lax
from jax.experimental import pallas as pl
from jax.experimental.pallas import tpu as pltpu
from jax.experimental.pallas import tpu_sc as plsc

NU = 10000
NI = 10000
N = NU + NI
D = 128
H = 64
B = 4096
E = 320000
EH = 160000
IB = 64
IB2 = 32
GCN_LAYERS = 3
TEMP = 0.2
REG = 1e-4
SSL = 0.1
IBL = 0.01
INTRA = 0.1


NC = 2
NS = 16
NP = 20480
NPU = 10240
EP = 327680
EHP = 163840
CE = 512
NCH_E = EP // NS // CE
NCH_H = EHP // NS // CE
RT = NP // NS
RTU = NPU // NS
EC = 64
CEG = 256
NCHG = EP // NS // CEG


def _sc_mesh():
    return plsc.VectorSubcoreMesh(core_axis_name="c", subcore_axis_name="s")


def _sc_params():
    return pltpu.CompilerParams(needs_layout_passes=False,
                                use_tc_tiling_on_sc=False)


def _zero_1d(ref, nwords):
    def z(i, _):
        ref[pl.ds(i * 16, 16)] = jnp.zeros((16,), jnp.float32)
        return _
    lax.fori_loop(0, nwords // 16, z, None)


def _rsqrt16(d):
    i = plsc.bitcast(d, jnp.int32)
    y = plsc.bitcast(jnp.int32(0x5F3759DF) - (i >> 1), jnp.float32)
    for _ in range(3):
        y = y * (1.5 - 0.5 * d * y * y)
    return y


def _sc_degree(adj2):
    @functools.partial(
        pl.kernel, mesh=_sc_mesh(), compiler_params=_sc_params(),
        out_type=jax.ShapeDtypeStruct((2, NP), jnp.float32),
        scratch_types=[
            pltpu.VMEM((NP,), jnp.float32),
            pltpu.VMEM((CE,), jnp.int32),
            pltpu.VMEM_SHARED((NS, NP), jnp.float32),
            pltpu.VMEM((RT,), jnp.float32),
            pltpu.VMEM((RT,), jnp.float32),
        ],
    )
    def k(adj, rsrd, hist, idxb, shared, tmp, red):
        c = lax.axis_index("c")
        s = lax.axis_index("s")
        _zero_1d(hist, NP)
        ones = jnp.ones((16,), jnp.float32)

        def chunk(ch, _):
            off = (s * NCH_E + ch) * CE
            pltpu.sync_copy(adj.at[c, pl.ds(off, CE)], idxb)

            def vec(j, x):
                v = idxb[pl.ds(j * 16, 16)]
                plsc.addupdate_scatter(hist, [v], ones)
                return x
            lax.fori_loop(0, CE // 16, vec, None)
            return _
        lax.fori_loop(0, NCH_E, chunk, None)
        pltpu.sync_copy(hist, shared.at[s])
        plsc.subcore_barrier()

        _zero_1d(red, RT)
        for t in range(NS):
            pltpu.sync_copy(shared.at[t, pl.ds(s * RT, RT)], tmp)

            def acc(j, _):
                red[pl.ds(j * 16, 16)] = (red[pl.ds(j * 16, 16)]
                                          + tmp[pl.ds(j * 16, 16)])
                return _
            lax.fori_loop(0, RT // 16, acc, None)

        def rq(j, _):
            d = jnp.maximum(red[pl.ds(j * 16, 16)], 1.0)
            red[pl.ds(j * 16, 16)] = _rsqrt16(d)
            return _
        lax.fori_loop(0, RT // 16, rq, None)
        pltpu.sync_copy(red, rsrd.at[c, pl.ds(s * RT, RT)])

    return k(adj2)


def _sc_gcn3(xs0_flat, adj2, rsrd, zrows):
    outs = (jax.ShapeDtypeStruct((2, NP, H), jnp.float32),
            jax.ShapeDtypeStruct((2 * NP, H), jnp.float32))

    @functools.partial(
        pl.kernel, mesh=_sc_mesh(), compiler_params=_sc_params(),
        out_type=outs,
        scratch_types=[
            pltpu.VMEM_SHARED((NP, H), jnp.float32),
            pltpu.VMEM((CEG,), jnp.int32),
            pltpu.VMEM((CEG,), jnp.int32),
            pltpu.VMEM((CEG,), jnp.int32),
            pltpu.VMEM((CEG, H), jnp.float32),
            pltpu.VMEM((EC, H), jnp.float32),
            pltpu.VMEM((EC, H), jnp.float32),
            pltpu.VMEM((EC, H), jnp.float32),
            pltpu.VMEM((EC, H), jnp.float32),
            pltpu.VMEM((RT,), jnp.float32),
            pltpu.VMEM((RT,), jnp.float32),
            pltpu.SemaphoreType.DMA,
        ],
    )
    def k(xs0, adj, rr, zr, S, xsw, accum, srcb, dstb, idxg, rows, abuf,
          sbuf, xbuf, zbuf, rsb, rdb, sem):
        c = lax.axis_index("c")
        s = lax.axis_index("s")
        pltpu.sync_copy(zr, zbuf)
        pltpu.sync_copy(rr.at[0, pl.ds(s * RT, RT)], rsb)
        pltpu.sync_copy(rr.at[1, pl.ds(s * RT, RT)], rdb)
        for e in range(RT // EC):
            pltpu.sync_copy(zbuf, accum.at[pl.ds(s * RT + e * EC, EC)])
        plsc.subcore_barrier()

        coff = c * NP
        for l in range(GCN_LAYERS):
            src_tbl = xs0 if l == 0 else xsw

            def chunk(ch, _):
                off = (s * NCHG + ch) * CEG
                pltpu.sync_copy(adj.at[0, pl.ds(off, CEG)], srcb)
                pltpu.sync_copy(adj.at[1, pl.ds(off, CEG)], dstb)

                def mkidx(j, x):
                    idxg[pl.ds(j * 16, 16)] = srcb[pl.ds(j * 16, 16)] + coff
                    return x
                lax.fori_loop(0, CEG // 16, mkidx, None)
                pltpu.async_copy(src_tbl.at[idxg], rows, sem).wait()
                pltpu.sync_copy(rows, accum.at[dstb], add=True)
                return _
            lax.fori_loop(0, NCHG, chunk, None)
            plsc.subcore_barrier()

            for e in range(RT // EC):
                r0 = s * RT + e * EC
                pltpu.sync_copy(accum.at[pl.ds(r0, EC)], abuf)
                pltpu.sync_copy(zbuf, accum.at[pl.ds(r0, EC)])
                if l > 0:
                    pltpu.sync_copy(S.at[c, pl.ds(r0, EC)], sbuf)

                def rowfn(i, _):
                    rix = jnp.full((16,), e * EC + i, jnp.int32)
                    rdv = plsc.load_gather(rdb, [rix])
                    rsv = plsc.load_gather(rsb, [rix])
                    for j in range(H // 16):
                        a = abuf[i, pl.ds(j * 16, 16)]
                        xv = a * rdv
                        if l > 0:
                            sbuf[i, pl.ds(j * 16, 16)] = (
                                sbuf[i, pl.ds(j * 16, 16)] + xv)
                        else:
                            sbuf[i, pl.ds(j * 16, 16)] = xv
                        if l < GCN_LAYERS - 1:
                            xbuf[i, pl.ds(j * 16, 16)] = xv * rsv
                    return _
                lax.fori_loop(0, EC, rowfn, None)
                pltpu.sync_copy(sbuf, S.at[c, pl.ds(r0, EC)])
                if l < GCN_LAYERS - 1:
                    pltpu.sync_copy(xbuf, xsw.at[pl.ds(coff + r0, EC)])
            plsc.subcore_barrier()

    return k(xs0_flat, adj2, rsrd, zrows)[0]


def _sc_dots(hvl_flat, edges4):
    CD = 128
    NCHD = EHP // NS // CD

    @functools.partial(
        pl.kernel, mesh=_sc_mesh(), compiler_params=_sc_params(),
        out_type=jax.ShapeDtypeStruct((4, EHP), jnp.float32),
        scratch_types=[
            pltpu.VMEM((CD,), jnp.int32),
            pltpu.VMEM((CD,), jnp.int32),
            pltpu.VMEM((CD,), jnp.int32),
            pltpu.VMEM((CD, D), jnp.float32),
            pltpu.VMEM((CD, D), jnp.float32),
            pltpu.VMEM((CD,), jnp.float32),
            pltpu.SemaphoreType.DMA,
        ],
    )
    def k(hvl, edges, ew, srcb, dstb, idxg, hs, hd, ewb, sem):
        c = lax.axis_index("c")
        s = lax.axis_index("s")
        coff = c * NPU
        lane0 = lax.iota(jnp.int32, 16) == 0
        for kk in range(2):
            si = 2 * c + kk

            def chunk(ch, _):
                off = (s * NCHD + ch) * CD
                pltpu.sync_copy(edges.at[si, 0, pl.ds(off, CD)], srcb)
                pltpu.sync_copy(edges.at[si, 1, pl.ds(off, CD)], dstb)

                def mkidx(j, x):
                    idxg[pl.ds(j * 16, 16)] = srcb[pl.ds(j * 16, 16)] + coff
                    return x
                lax.fori_loop(0, CD // 16, mkidx, None)
                pltpu.async_copy(hvl.at[idxg], hs, sem).wait()

                def mkidx2(j, x):
                    idxg[pl.ds(j * 16, 16)] = dstb[pl.ds(j * 16, 16)] + coff
                    return x
                lax.fori_loop(0, CD // 16, mkidx2, None)
                pltpu.async_copy(hvl.at[idxg], hd, sem).wait()

                def dot1(i, x):
                    acc = hs[i, pl.ds(0, 16)] * hd[i, pl.ds(0, 16)]
                    for j in range(1, D // 16):
                        acc = acc + (hs[i, pl.ds(j * 16, 16)]
                                     * hd[i, pl.ds(j * 16, 16)])
                    dv = jnp.full((16,), jnp.sum(acc), jnp.float32)
                    plsc.store_scatter(ewb, [jnp.full((16,), i, jnp.int32)],
                                       dv, mask=lane0)
                    return x
                lax.fori_loop(0, CD, dot1, None)

                def sig(j, x):
                    v = ewb[pl.ds(j * 16, 16)]
                    ewb[pl.ds(j * 16, 16)] = 1.0 / (1.0 + jnp.exp(-v))
                    return x
                lax.fori_loop(0, CD // 16, sig, None)
                pltpu.sync_copy(ewb, ew.at[si, pl.ds(off, CD)])
                return _
            lax.fori_loop(0, NCHD, chunk, None)

    return k(hvl_flat, edges4)


def _sc_seg(tbl_flat, edges4, ew4, zrows, tsel):
    CS = 512
    NCHS = EHP // NS // CS

    @functools.partial(
        pl.kernel, mesh=_sc_mesh(), compiler_params=_sc_params(),
        out_type=jax.ShapeDtypeStruct((4, 2, NPU, H), jnp.float32),
        scratch_types=[
            pltpu.VMEM_SHARED((NPU, H), jnp.float32),
            pltpu.VMEM((CS,), jnp.int32),
            pltpu.VMEM((CS,), jnp.int32),
            pltpu.VMEM((CS,), jnp.int32),
            pltpu.VMEM((CS,), jnp.float32),
            pltpu.VMEM((CS, H), jnp.float32),
            pltpu.VMEM((EC, H), jnp.float32),
            pltpu.SemaphoreType.DMA,
        ],
    )
    def k(tbl, edges, ew, zr, out, accum, srcb, dstb, idxg, ewb, rows,
          zbuf, sem):
        c = lax.axis_index("c")
        s = lax.axis_index("s")
        pltpu.sync_copy(zr, zbuf)
        for ae in range(4):
            base = tsel[ae] * 2 * NPU
            for e in range(RTU // EC):
                pltpu.sync_copy(zbuf, accum.at[pl.ds(s * RTU + e * EC, EC)])
            plsc.subcore_barrier()

            def chunk(ch, x):
                off = (s * NCHS + ch) * CS
                pltpu.sync_copy(edges.at[ae, 0, pl.ds(off, CS)], srcb)
                pltpu.sync_copy(edges.at[ae, 1, pl.ds(off, CS)], dstb)
                pltpu.sync_copy(ew.at[ae, pl.ds(off, CS)], ewb)

                def mkidx(j, y):
                    idxg[pl.ds(j * 16, 16)] = (srcb[pl.ds(j * 16, 16)]
                                               + (base + c * NPU))
                    return y
                lax.fori_loop(0, CS // 16, mkidx, None)
                pltpu.async_copy(tbl.at[idxg], rows, sem).wait()

                def wrow(i, y):
                    wv = plsc.load_gather(
                        ewb, [jnp.full((16,), i, jnp.int32)])
                    for j in range(H // 16):
                        rows[i, pl.ds(j * 16, 16)] = (
                            rows[i, pl.ds(j * 16, 16)] * wv)
                    return y
                lax.fori_loop(0, CS, wrow, None)
                pltpu.sync_copy(rows, accum.at[dstb], add=True)
                return x
            lax.fori_loop(0, NCHS, chunk, None)
            plsc.subcore_barrier()
            for e in range(RTU // EC):
                r0 = s * RTU + e * EC
                pltpu.sync_copy(accum.at[pl.ds(r0, EC)],
                                out.at[ae, c, pl.ds(r0, EC)])

    return k(tbl_flat, edges4, ew4, zrows)


def _sc_bgather(S_flat, h_flat, ego_flat, user, positive, negative):
    BW = B // (NC * NS)

    outs = (jax.ShapeDtypeStruct((2, 3, B, H), jnp.float32),
            jax.ShapeDtypeStruct((4, B, D), jnp.float32),
            jax.ShapeDtypeStruct((3, B, D), jnp.float32))

    @functools.partial(
        pl.kernel, mesh=_sc_mesh(), compiler_params=_sc_params(),
        out_type=outs,
        scratch_types=[
            pltpu.VMEM((BW,), jnp.int32),
            pltpu.VMEM((BW,), jnp.int32),
            pltpu.VMEM((BW,), jnp.int32),
            pltpu.VMEM((BW,), jnp.int32),
            pltpu.VMEM((BW, H), jnp.float32),
            pltpu.VMEM((BW, D), jnp.float32),
            pltpu.SemaphoreType.DMA,
        ],
    )
    def k(Sf, hf, ef, iu, ip, inn, oS, oh, oe, idxu, idxp, idxn, idxg,
          rows64, rows128, sem):
        c = lax.axis_index("c")
        s = lax.axis_index("s")
        w = s * NC + c
        b0 = w * BW
        pltpu.sync_copy(iu.at[pl.ds(b0, BW)], idxu)
        pltpu.sync_copy(ip.at[pl.ds(b0, BW)], idxp)
        pltpu.sync_copy(inn.at[pl.ds(b0, BW)], idxn)

        def mkidx(src_ref, off):
            def go(j, x):
                idxg[pl.ds(j * 16, 16)] = src_ref[pl.ds(j * 16, 16)] + off
                return x
            lax.fori_loop(0, BW // 16, go, None)

        for half in range(2):
            for slot, (iref, noff) in enumerate(
                    [(idxu, 0), (idxp, NU), (idxn, NU)]):
                mkidx(iref, half * NP + noff)
                pltpu.async_copy(Sf.at[idxg], rows64, sem).wait()
                pltpu.sync_copy(rows64, oS.at[half, slot, pl.ds(b0, BW)])
        for ae, iref in enumerate([idxu, idxu, idxp, idxp]):
            mkidx(iref, ae * NPU)
            pltpu.async_copy(hf.at[idxg], rows128, sem).wait()
            pltpu.sync_copy(rows128, oh.at[ae, pl.ds(b0, BW)])
        for slot, (iref, noff) in enumerate(
                [(idxu, 0), (idxp, NPU), (idxn, NPU)]):
            mkidx(iref, noff)
            pltpu.async_copy(ef.at[idxg], rows128, sem).wait()
            pltpu.sync_copy(rows128, oe.at[slot, pl.ds(b0, BW)])

    return k(S_flat, h_flat, ego_flat, user, positive, negative)


_TCB = 512


def _tc_prescale(x0p, rsrd):
    def body(x_ref, r_ref, o_ref):
        o_ref[...] = (x_ref[...] * r_ref[0, :, None])[None]

    out = pl.pallas_call(
        body,
        grid=(2, NP // _TCB),
        in_specs=[
            pl.BlockSpec((_TCB, H), lambda c, i: (i, c)),
            pl.BlockSpec((1, _TCB), lambda c, i: (0, i)),
        ],
        out_specs=pl.BlockSpec((1, _TCB, H), lambda c, i: (c, i, 0)),
        out_shape=jax.ShapeDtypeStruct((2, NP, H), jnp.float32),
    )(x0p, rsrd)
    return out.reshape(2 * NP, H)


def _tc_hvl(tabs, W_vl):
    def body(x_ref, w_ref, o_ref):
        o_ref[...] = jnp.dot(x_ref[0], w_ref[...],
                             preferred_element_type=jnp.float32)[None]

    out = pl.pallas_call(
        body,
        grid=(2, NPU // _TCB),
        in_specs=[
            pl.BlockSpec((1, _TCB, D), lambda t, i: (t, i, 0)),
            pl.BlockSpec((D, D), lambda t, i: (0, 0)),
        ],
        out_specs=pl.BlockSpec((1, _TCB, D), lambda t, i: (t, i, 0)),
        out_shape=jax.ShapeDtypeStruct((2, NPU, D), jnp.float32),
    )(tabs, W_vl)
    return out.reshape(2 * NPU, D)


def _tc_mid(h1, W1s):
    def body(h_ref, w_ref, o_ref):
        x = jnp.concatenate([h_ref[0, 0], h_ref[0, 1]], axis=1)
        y = jnp.maximum(jnp.dot(x, w_ref[0],
                                preferred_element_type=jnp.float32), 0.0)
        o_ref[0, 0] = y[:, :H]
        o_ref[0, 1] = y[:, H:]

    return pl.pallas_call(
        body,
        grid=(4, NPU // _TCB),
        in_specs=[
            pl.BlockSpec((1, 2, _TCB, H), lambda a, i: (a, 0, i, 0)),
            pl.BlockSpec((1, D, D), lambda a, i: (a, 0, 0)),
        ],
        out_specs=pl.BlockSpec((1, 2, _TCB, H), lambda a, i: (a, 0, i, 0)),
        out_shape=jax.ShapeDtypeStruct((4, 2, NPU, H), jnp.float32),
    )(h1, W1s)


def _tc_out(h2, W2s):
    def body(h_ref, w_ref, o_ref):
        x = jnp.concatenate([h_ref[0, 0], h_ref[0, 1]], axis=1)
        o_ref[...] = jnp.dot(x, w_ref[0],
                             preferred_element_type=jnp.float32)[None]

    return pl.pallas_call(
        body,
        grid=(4, NPU // _TCB),
        in_specs=[
            pl.BlockSpec((1, 2, _TCB, H), lambda a, i: (a, 0, i, 0)),
            pl.BlockSpec((1, D, D), lambda a, i: (a, 0, 0)),
        ],
        out_specs=pl.BlockSpec((1, _TCB, D), lambda a, i: (a, i, 0)),
        out_shape=jax.ShapeDtypeStruct((4, NPU, D), jnp.float32),
    )(h2, W2s)


def _tc_info_nce(a1, b1, a2, b2, alpha):
    RB = 512

    def body(a1_ref, b1_ref, a2_ref, b2_ref, o_ref, z1s, z2s, accs):
        z1 = alpha * a1_ref[...] + b1_ref[...]
        z2 = alpha * a2_ref[...] + b2_ref[...]
        z1 = z1 / (jnp.sqrt(jnp.sum(z1 * z1, axis=1, keepdims=True)) + 1e-8)
        z2 = z2 / (jnp.sqrt(jnp.sum(z2 * z2, axis=1, keepdims=True)) + 1e-8)
        z1s[...] = z1
        z2s[...] = z2

        def blk(i, acc):
            zb = z1s[pl.ds(i * RB, RB), :]
            sc = lax.dot_general(zb, z2s[...],
                                 (((1,), (1,)), ((), ())),
                                 preferred_element_type=jnp.float32) / TEMP
            m = jnp.max(sc, axis=1, keepdims=True)
            lse = jnp.log(jnp.sum(jnp.exp(sc - m), axis=1)) + m[:, 0]
            pos = jnp.sum(zb * z2s[pl.ds(i * RB, RB), :], axis=1) / TEMP
            return acc + jnp.sum(lse - pos)

        tot = lax.fori_loop(0, B // RB, blk, jnp.float32(0.0))
        o_ref[...] = jnp.full((8, 128), tot / B, jnp.float32)

    out = pl.pallas_call(
        body,
        out_shape=jax.ShapeDtypeStruct((8, 128), jnp.float32),
        scratch_shapes=[
            pltpu.VMEM((B, D), jnp.float32),
            pltpu.VMEM((B, D), jnp.float32),
            pltpu.VMEM((1, 1), jnp.float32),
        ],
    )(a1, b1, a2, b2)
    return out[0, 0]


def _tc_bpr_reg(ueS, peS, neS, ge):
    def body(u_ref, p_ref, n_ref, e_ref, o_ref):
        ue = u_ref[...] / 3.0
        pe = p_ref[...] / 3.0
        ne = n_ref[...] / 3.0
        d = jnp.sum(ue * pe, axis=1) - jnp.sum(ue * ne, axis=1)
        bpr = jnp.mean(jnp.where(
            d > 0,
            -jnp.log1p(jnp.exp(-d)),
            d - jnp.log1p(jnp.exp(d))))
        reg = REG * 0.5 * jnp.sum(e_ref[...] ** 2) / B
        o_ref[...] = jnp.concatenate(
            [jnp.full((1, 128), -bpr, jnp.float32),
             jnp.full((1, 128), reg, jnp.float32),
             jnp.zeros((6, 128), jnp.float32)], axis=0)

    out = pl.pallas_call(
        body,
        out_shape=jax.ShapeDtypeStruct((8, 128), jnp.float32),
    )(ueS, peS, neS, ge)
    return out[:2, 0]


def _tc_kl(hu0, hu1, hi0, hi1, Wuc, buc, Wic, bic):
    def body(u0, u1, i0, i1, wu, bu, wi, bi, o_ref):
        mask = (lax.broadcasted_iota(jnp.int32, (NPU, 1), 0) < NU)

        def klterm(h0, h1, w, b, nn_):
            node = jnp.dot(0.5 * (h0[...] + h1[...]), w[...],
                           preferred_element_type=jnp.float32) + b[0]
            mu = node[:, :IB2]
            st = jnp.logaddexp(node[:, IB2:] - IB2, 0.0)
            st = jnp.where(mask, st, 1.0)
            mu = jnp.where(mask, mu, 0.0)
            t = 1.0 + 2.0 * jnp.log(st) - mu * mu - st * st
            return -0.5 * (jnp.sum(t) / nn_) / jnp.log(2.0)

        ukl = klterm(u0, u1, wu, bu, NU)
        ikl = klterm(i0, i1, wi, bi, NI)
        o_ref[...] = jnp.concatenate(
            [jnp.full((1, 128), ukl, jnp.float32),
             jnp.full((1, 128), ikl, jnp.float32),
             jnp.zeros((6, 128), jnp.float32)], axis=0)

    out = pl.pallas_call(
        body,
        out_shape=jax.ShapeDtypeStruct((8, 128), jnp.float32),
    )(hu0, hu1, hi0, hi1, Wuc, buc[None, :], Wic, bic[None, :])
    return out[:2, 0]


def _info_nce(z1, z2, t):
    z1 = z1 / (jnp.linalg.norm(z1, axis=1, keepdims=True) + 1e-8)
    z2 = z2 / (jnp.linalg.norm(z2, axis=1, keepdims=True) + 1e-8)
    pos = jnp.sum(z1 * z2, axis=1) / t
    ttl = (z1 @ z2.T) / t
    return jnp.mean(jax.scipy.special.logsumexp(ttl, axis=1) - pos)


def _stack5_kernel(x_ref, o_ref):
    o_ref[...] = x_ref[...]


def _stack5(vals):
    x = jnp.broadcast_to(jnp.pad(jnp.stack(vals), (0, 3))[:, None], (8, 128))
    out = pl.pallas_call(
        _stack5_kernel,
        out_shape=jax.ShapeDtypeStruct((8, 128), jnp.float32),
    )(x)
    return out[:5, 0]


def kernel(user_table, item_table, W_vl, Wu0a, Wu0b, Wu1a, Wu1b, Wi0a, Wi0b,
           Wi1a, Wi1b, Wuc, buc, Wic, bic, adj_src, adj_dst, uu0_src, uu0_dst,
           uu1_src, uu1_dst, ii0_src, ii0_dst, ii1_src, ii1_dst, user,
           positive, negative):
    npad_e = EP - E
    adj2 = jnp.stack([
        jnp.concatenate([adj_src, jnp.full((npad_e,), N, jnp.int32)]),
        jnp.concatenate([adj_dst, jnp.full((npad_e,), N, jnp.int32)]),
    ])
    rsrd = _sc_degree(adj2)
    rs = rsrd[0]

    x0 = jnp.concatenate([user_table, item_table], axis=0)
    x0p = jnp.pad(x0, ((0, NP - N), (0, 0)))
    xs0 = x0p * rs[:, None]
    xs0_flat = jnp.concatenate([xs0[:, :H], xs0[:, H:]], axis=0)
    zrows = jnp.zeros((EC, H), jnp.float32)
    S2 = _sc_gcn3(xs0_flat, adj2, rsrd, zrows)
    S = jnp.concatenate([S2[0, :N], S2[1, :N]], axis=1)

    def view_learner(src, dst, emb):
        h = emb @ W_vl
        return jax.nn.sigmoid(jnp.sum(h[src] * h[dst], axis=1))

    def autoenc(src, dst, ew, emb, W1, W2, nn_):
        h = jax.ops.segment_sum(emb[src] * ew[:, None], dst, num_segments=nn_)
        h = jax.nn.relu(h @ W1)
        h2 = jax.ops.segment_sum(h[src] * ew[:, None], dst, num_segments=nn_)
        return h2 @ W2

    hu0 = autoenc(uu0_src, uu0_dst,
                  view_learner(uu0_src, uu0_dst, user_table),
                  user_table, Wu0a, Wu0b, NU)
    hu1 = autoenc(uu1_src, uu1_dst,
                  view_learner(uu1_src, uu1_dst, user_table),
                  user_table, Wu1a, Wu1b, NU)
    hi0 = autoenc(ii0_src, ii0_dst,
                  view_learner(ii0_src, ii0_dst, item_table),
                  item_table, Wi0a, Wi0b, NI)
    hi1 = autoenc(ii1_src, ii1_dst,
                  view_learner(ii1_src, ii1_dst, item_table),
                  item_table, Wi1a, Wi1b, NI)

    user_node = 0.5 * (hu0 + hu1) @ Wuc + buc
    item_node = 0.5 * (hi0 + hi1) @ Wic + bic
    user_mu = user_node[:, :IB2]
    user_std = jax.nn.softplus(user_node[:, IB2:] - IB2)
    item_mu = item_node[:, :IB2]
    item_std = jax.nn.softplus(item_node[:, IB2:] - IB2)
    ukl = -0.5 * jnp.mean(jnp.sum(
        1.0 + 2.0 * jnp.log(user_std) - user_mu ** 2 - user_std ** 2,
        axis=1)) / jnp.log(2.0)
    ikl = -0.5 * jnp.mean(jnp.sum(
        1.0 + 2.0 * jnp.log(item_std) - item_mu ** 2 - item_std ** 2,
        axis=1)) / jnp.log(2.0)
    ib_loss = IBL * (ukl + ikl)

    final = S / 3.0
    ue = final[user]
    pe = final[NU + positive]
    ne = final[NU + negative]
    g_hu0 = hu0[user]
    g_hu1 = hu1[user]
    g_hi0 = hi0[positive]
    g_hi1 = hi1[positive]
    ego_u = user_table[user]
    ego_p = item_table[positive]
    ego_n = item_table[negative]

    intra_loss = INTRA * (_info_nce(g_hu0, g_hu1, TEMP)
                          + _info_nce(g_hi0, g_hi1, TEMP))
    bpr_loss = -jnp.mean(jax.nn.log_sigmoid(
        jnp.sum(ue * pe, axis=1) - jnp.sum(ue * ne, axis=1)))
    reg_loss = REG * 0.5 * (jnp.sum(ego_u ** 2) + jnp.sum(ego_p ** 2)
                            + jnp.sum(ego_n ** 2)) / B
    ssl_loss = SSL * (_info_nce(ue + g_hu0, ue + g_hu1, TEMP)
                      + _info_nce(pe + g_hi0, pe + g_hi1, TEMP))
    return _stack5([bpr_loss, reg_loss, ssl_loss, intra_loss, ib_loss])


# speedup vs baseline: 2.3945x; 1.4964x over previous
import functools

import jax
import jax.numpy as jnp
from jax import lax
from jax.experimental import pallas as pl
from jax.experimental.pallas import tpu as pltpu
from jax.experimental.pallas import tpu_sc as plsc

NU = 10000
NI = 10000
N = NU + NI
D = 128
H = 64
B = 4096
E = 320000
EH = 160000
IB = 64
IB2 = 32
GCN_LAYERS = 3
TEMP = 0.2
REG = 1e-4
SSL = 0.1
IBL = 0.01
INTRA = 0.1


NC = 2
NS = 16
NP = 20480
NPU = 10240
EP = 327680
EHP = 163840
CE = 512
NCH_E = EP // NS // CE
NCH_H = EHP // NS // CE
RT = NP // NS
RTU = NPU // NS
EC = 64
CEG = 256
NCHG = EP // NS // CEG


def _sc_mesh():
    return plsc.VectorSubcoreMesh(core_axis_name="c", subcore_axis_name="s")


def _sc_params():
    return pltpu.CompilerParams(needs_layout_passes=False,
                                use_tc_tiling_on_sc=False)


def _zero_1d(ref, nwords):
    def z(i, _):
        ref[pl.ds(i * 16, 16)] = jnp.zeros((16,), jnp.float32)
        return _
    lax.fori_loop(0, nwords // 16, z, None)


def _rsqrt16(d):
    i = plsc.bitcast(d, jnp.int32)
    y = plsc.bitcast(jnp.int32(0x5F3759DF) - (i >> 1), jnp.float32)
    for _ in range(3):
        y = y * (1.5 - 0.5 * d * y * y)
    return y


def _sc_degree(adj2):
    @functools.partial(
        pl.kernel, mesh=_sc_mesh(), compiler_params=_sc_params(),
        out_type=jax.ShapeDtypeStruct((2, NP), jnp.float32),
        scratch_types=[
            pltpu.VMEM((NP,), jnp.float32),
            pltpu.VMEM((CE,), jnp.int32),
            pltpu.VMEM_SHARED((NS, NP), jnp.float32),
            pltpu.VMEM((RT,), jnp.float32),
            pltpu.VMEM((RT,), jnp.float32),
        ],
    )
    def k(adj, rsrd, hist, idxb, shared, tmp, red):
        c = lax.axis_index("c")
        s = lax.axis_index("s")
        _zero_1d(hist, NP)
        ones = jnp.ones((16,), jnp.float32)

        def chunk(ch, _):
            off = (s * NCH_E + ch) * CE
            pltpu.sync_copy(adj.at[c, pl.ds(off, CE)], idxb)

            def vec(j, x):
                v = idxb[pl.ds(j * 16, 16)]
                plsc.addupdate_scatter(hist, [v], ones)
                return x
            lax.fori_loop(0, CE // 16, vec, None)
            return _
        lax.fori_loop(0, NCH_E, chunk, None)
        pltpu.sync_copy(hist, shared.at[s])
        plsc.subcore_barrier()

        _zero_1d(red, RT)
        for t in range(NS):
            pltpu.sync_copy(shared.at[t, pl.ds(s * RT, RT)], tmp)

            def acc(j, _):
                red[pl.ds(j * 16, 16)] = (red[pl.ds(j * 16, 16)]
                                          + tmp[pl.ds(j * 16, 16)])
                return _
            lax.fori_loop(0, RT // 16, acc, None)

        def rq(j, _):
            d = jnp.maximum(red[pl.ds(j * 16, 16)], 1.0)
            red[pl.ds(j * 16, 16)] = _rsqrt16(d)
            return _
        lax.fori_loop(0, RT // 16, rq, None)
        pltpu.sync_copy(red, rsrd.at[c, pl.ds(s * RT, RT)])

    return k(adj2)


def _sc_gcn3(xs0_flat, adj2, rsrd, zrows):
    outs = (jax.ShapeDtypeStruct((2, NP, H), jnp.float32),
            jax.ShapeDtypeStruct((2 * NP, H), jnp.float32))

    @functools.partial(
        pl.kernel, mesh=_sc_mesh(), compiler_params=_sc_params(),
        out_type=outs,
        scratch_types=[
            pltpu.VMEM_SHARED((NP, H), jnp.float32),
            pltpu.VMEM((CEG,), jnp.int32),
            pltpu.VMEM((CEG,), jnp.int32),
            pltpu.VMEM((CEG,), jnp.int32),
            pltpu.VMEM((CEG, H), jnp.float32),
            pltpu.VMEM((EC, H), jnp.float32),
            pltpu.VMEM((EC, H), jnp.float32),
            pltpu.VMEM((EC, H), jnp.float32),
            pltpu.VMEM((EC, H), jnp.float32),
            pltpu.VMEM((RT,), jnp.float32),
            pltpu.VMEM((RT,), jnp.float32),
            pltpu.SemaphoreType.DMA,
        ],
    )
    def k(xs0, adj, rr, zr, S, xsw, accum, srcb, dstb, idxg, rows, abuf,
          sbuf, xbuf, zbuf, rsb, rdb, sem):
        c = lax.axis_index("c")
        s = lax.axis_index("s")
        pltpu.sync_copy(zr, zbuf)
        pltpu.sync_copy(rr.at[0, pl.ds(s * RT, RT)], rsb)
        pltpu.sync_copy(rr.at[1, pl.ds(s * RT, RT)], rdb)
        for e in range(RT // EC):
            pltpu.sync_copy(zbuf, accum.at[pl.ds(s * RT + e * EC, EC)])
        plsc.subcore_barrier()

        coff = c * NP
        for l in range(GCN_LAYERS):
            src_tbl = xs0 if l == 0 else xsw

            def chunk(ch, _):
                off = (s * NCHG + ch) * CEG
                pltpu.sync_copy(adj.at[0, pl.ds(off, CEG)], srcb)
                pltpu.sync_copy(adj.at[1, pl.ds(off, CEG)], dstb)

                def mkidx(j, x):
                    idxg[pl.ds(j * 16, 16)] = srcb[pl.ds(j * 16, 16)] + coff
                    return x
                lax.fori_loop(0, CEG // 16, mkidx, None)
                pltpu.async_copy(src_tbl.at[idxg], rows, sem).wait()
                pltpu.sync_copy(rows, accum.at[dstb], add=True)
                return _
            lax.fori_loop(0, NCHG, chunk, None)
            plsc.subcore_barrier()

            for e in range(RT // EC):
                r0 = s * RT + e * EC
                pltpu.sync_copy(accum.at[pl.ds(r0, EC)], abuf)
                pltpu.sync_copy(zbuf, accum.at[pl.ds(r0, EC)])
                if l > 0:
                    pltpu.sync_copy(S.at[c, pl.ds(r0, EC)], sbuf)

                def rowfn(i, _):
                    rix = jnp.full((16,), e * EC + i, jnp.int32)
                    rdv = plsc.load_gather(rdb, [rix])
                    rsv = plsc.load_gather(rsb, [rix])
                    for j in range(H // 16):
                        a = abuf[i, pl.ds(j * 16, 16)]
                        xv = a * rdv
                        if l > 0:
                            sbuf[i, pl.ds(j * 16, 16)] = (
                                sbuf[i, pl.ds(j * 16, 16)] + xv)
                        else:
                            sbuf[i, pl.ds(j * 16, 16)] = xv
                        if l < GCN_LAYERS - 1:
                            xbuf[i, pl.ds(j * 16, 16)] = xv * rsv
                    return _
                lax.fori_loop(0, EC, rowfn, None)
                pltpu.sync_copy(sbuf, S.at[c, pl.ds(r0, EC)])
                if l < GCN_LAYERS - 1:
                    pltpu.sync_copy(xbuf, xsw.at[pl.ds(coff + r0, EC)])
            plsc.subcore_barrier()

    return k(xs0_flat, adj2, rsrd, zrows)[0]


def _sc_dots(hvl_flat, edges4):
    CD = 128
    NCHD = EHP // NS // CD

    @functools.partial(
        pl.kernel, mesh=_sc_mesh(), compiler_params=_sc_params(),
        out_type=jax.ShapeDtypeStruct((4, EHP), jnp.float32),
        scratch_types=[
            pltpu.VMEM((CD,), jnp.int32),
            pltpu.VMEM((CD,), jnp.int32),
            pltpu.VMEM((CD,), jnp.int32),
            pltpu.VMEM((CD, D), jnp.float32),
            pltpu.VMEM((CD, D), jnp.float32),
            pltpu.VMEM((CD,), jnp.float32),
            pltpu.SemaphoreType.DMA,
        ],
    )
    def k(hvl, edges, ew, srcb, dstb, idxg, hs, hd, ewb, sem):
        c = lax.axis_index("c")
        s = lax.axis_index("s")
        coff = c * NPU
        lane0 = lax.iota(jnp.int32, 16) == 0
        for kk in range(2):
            si = 2 * c + kk

            def chunk(ch, _):
                off = (s * NCHD + ch) * CD
                pltpu.sync_copy(edges.at[si, 0, pl.ds(off, CD)], srcb)
                pltpu.sync_copy(edges.at[si, 1, pl.ds(off, CD)], dstb)

                def mkidx(j, x):
                    idxg[pl.ds(j * 16, 16)] = srcb[pl.ds(j * 16, 16)] + coff
                    return x
                lax.fori_loop(0, CD // 16, mkidx, None)
                pltpu.async_copy(hvl.at[idxg], hs, sem).wait()

                def mkidx2(j, x):
                    idxg[pl.ds(j * 16, 16)] = dstb[pl.ds(j * 16, 16)] + coff
                    return x
                lax.fori_loop(0, CD // 16, mkidx2, None)
                pltpu.async_copy(hvl.at[idxg], hd, sem).wait()

                def dot1(i, x):
                    acc = hs[i, pl.ds(0, 16)] * hd[i, pl.ds(0, 16)]
                    for j in range(1, D // 16):
                        acc = acc + (hs[i, pl.ds(j * 16, 16)]
                                     * hd[i, pl.ds(j * 16, 16)])
                    dv = jnp.full((16,), jnp.sum(acc), jnp.float32)
                    plsc.store_scatter(ewb, [jnp.full((16,), i, jnp.int32)],
                                       dv, mask=lane0)
                    return x
                lax.fori_loop(0, CD, dot1, None)

                def sig(j, x):
                    v = ewb[pl.ds(j * 16, 16)]
                    ewb[pl.ds(j * 16, 16)] = 1.0 / (1.0 + jnp.exp(-v))
                    return x
                lax.fori_loop(0, CD // 16, sig, None)
                pltpu.sync_copy(ewb, ew.at[si, pl.ds(off, CD)])
                return _
            lax.fori_loop(0, NCHD, chunk, None)

    return k(hvl_flat, edges4)


def _sc_seg(tbl_flat, edges4, ew4, zrows, tsel):
    CS = 512
    NCHS = EHP // NS // CS

    @functools.partial(
        pl.kernel, mesh=_sc_mesh(), compiler_params=_sc_params(),
        out_type=jax.ShapeDtypeStruct((4, 2, NPU, H), jnp.float32),
        scratch_types=[
            pltpu.VMEM_SHARED((NPU, H), jnp.float32),
            pltpu.VMEM((CS,), jnp.int32),
            pltpu.VMEM((CS,), jnp.int32),
            pltpu.VMEM((CS,), jnp.int32),
            pltpu.VMEM((CS,), jnp.float32),
            pltpu.VMEM((CS, H), jnp.float32),
            pltpu.VMEM((EC, H), jnp.float32),
            pltpu.SemaphoreType.DMA,
        ],
    )
    def k(tbl, edges, ew, zr, out, accum, srcb, dstb, idxg, ewb, rows,
          zbuf, sem):
        c = lax.axis_index("c")
        s = lax.axis_index("s")
        pltpu.sync_copy(zr, zbuf)
        for ae in range(4):
            base = tsel[ae] * 2 * NPU
            for e in range(RTU // EC):
                pltpu.sync_copy(zbuf, accum.at[pl.ds(s * RTU + e * EC, EC)])
            plsc.subcore_barrier()

            def chunk(ch, x):
                off = (s * NCHS + ch) * CS
                pltpu.sync_copy(edges.at[ae, 0, pl.ds(off, CS)], srcb)
                pltpu.sync_copy(edges.at[ae, 1, pl.ds(off, CS)], dstb)
                pltpu.sync_copy(ew.at[ae, pl.ds(off, CS)], ewb)

                def mkidx(j, y):
                    idxg[pl.ds(j * 16, 16)] = (srcb[pl.ds(j * 16, 16)]
                                               + (base + c * NPU))
                    return y
                lax.fori_loop(0, CS // 16, mkidx, None)
                pltpu.async_copy(tbl.at[idxg], rows, sem).wait()

                def wrow(i, y):
                    wv = plsc.load_gather(
                        ewb, [jnp.full((16,), i, jnp.int32)])
                    for j in range(H // 16):
                        rows[i, pl.ds(j * 16, 16)] = (
                            rows[i, pl.ds(j * 16, 16)] * wv)
                    return y
                lax.fori_loop(0, CS, wrow, None)
                pltpu.sync_copy(rows, accum.at[dstb], add=True)
                return x
            lax.fori_loop(0, NCHS, chunk, None)
            plsc.subcore_barrier()
            for e in range(RTU // EC):
                r0 = s * RTU + e * EC
                pltpu.sync_copy(accum.at[pl.ds(r0, EC)],
                                out.at[ae, c, pl.ds(r0, EC)])

    return k(tbl_flat, edges4, ew4, zrows)


def _sc_bgather(S_flat, h_flat, ego_flat, user, positive, negative):
    BW = B // (NC * NS)

    outs = (jax.ShapeDtypeStruct((2, 3, B, H), jnp.float32),
            jax.ShapeDtypeStruct((4, B, D), jnp.float32),
            jax.ShapeDtypeStruct((3, B, D), jnp.float32))

    @functools.partial(
        pl.kernel, mesh=_sc_mesh(), compiler_params=_sc_params(),
        out_type=outs,
        scratch_types=[
            pltpu.VMEM((BW,), jnp.int32),
            pltpu.VMEM((BW,), jnp.int32),
            pltpu.VMEM((BW,), jnp.int32),
            pltpu.VMEM((BW,), jnp.int32),
            pltpu.VMEM((BW, H), jnp.float32),
            pltpu.VMEM((BW, D), jnp.float32),
            pltpu.SemaphoreType.DMA,
        ],
    )
    def k(Sf, hf, ef, iu, ip, inn, oS, oh, oe, idxu, idxp, idxn, idxg,
          rows64, rows128, sem):
        c = lax.axis_index("c")
        s = lax.axis_index("s")
        w = s * NC + c
        b0 = w * BW
        pltpu.sync_copy(iu.at[pl.ds(b0, BW)], idxu)
        pltpu.sync_copy(ip.at[pl.ds(b0, BW)], idxp)
        pltpu.sync_copy(inn.at[pl.ds(b0, BW)], idxn)

        def mkidx(src_ref, off):
            def go(j, x):
                idxg[pl.ds(j * 16, 16)] = src_ref[pl.ds(j * 16, 16)] + off
                return x
            lax.fori_loop(0, BW // 16, go, None)

        for half in range(2):
            for slot, (iref, noff) in enumerate(
                    [(idxu, 0), (idxp, NU), (idxn, NU)]):
                mkidx(iref, half * NP + noff)
                pltpu.async_copy(Sf.at[idxg], rows64, sem).wait()
                pltpu.sync_copy(rows64, oS.at[half, slot, pl.ds(b0, BW)])
        for ae, iref in enumerate([idxu, idxu, idxp, idxp]):
            mkidx(iref, ae * NPU)
            pltpu.async_copy(hf.at[idxg], rows128, sem).wait()
            pltpu.sync_copy(rows128, oh.at[ae, pl.ds(b0, BW)])
        for slot, (iref, noff) in enumerate(
                [(idxu, 0), (idxp, NPU), (idxn, NPU)]):
            mkidx(iref, noff)
            pltpu.async_copy(ef.at[idxg], rows128, sem).wait()
            pltpu.sync_copy(rows128, oe.at[slot, pl.ds(b0, BW)])

    return k(S_flat, h_flat, ego_flat, user, positive, negative)


_TCB = 512


def _tc_prescale(x0p, rsrd):
    def body(x_ref, r_ref, o_ref):
        rs = r_ref[...][0][:, None]
        xs = x_ref[...] * rs
        o_ref[0] = xs[:, :H]
        o_ref[1] = xs[:, H:]

    out = pl.pallas_call(
        body,
        grid=(NP // _TCB,),
        in_specs=[
            pl.BlockSpec((_TCB, D), lambda i: (i, 0)),
            pl.BlockSpec((2, _TCB), lambda i: (0, i)),
        ],
        out_specs=pl.BlockSpec((2, _TCB, H), lambda i: (0, i, 0)),
        out_shape=jax.ShapeDtypeStruct((2, NP, H), jnp.float32),
    )(x0p, rsrd)
    return out.reshape(2 * NP, H)


def _tc_hvl(tabs, W_vl):
    def body(x_ref, w_ref, o_ref):
        o_ref[...] = jnp.dot(x_ref[0], w_ref[...],
                             preferred_element_type=jnp.float32)[None]

    out = pl.pallas_call(
        body,
        grid=(2, NPU // _TCB),
        in_specs=[
            pl.BlockSpec((1, _TCB, D), lambda t, i: (t, i, 0)),
            pl.BlockSpec((D, D), lambda t, i: (0, 0)),
        ],
        out_specs=pl.BlockSpec((1, _TCB, D), lambda t, i: (t, i, 0)),
        out_shape=jax.ShapeDtypeStruct((2, NPU, D), jnp.float32),
    )(tabs, W_vl)
    return out.reshape(2 * NPU, D)


def _tc_mid(h1, W1s):
    def body(h_ref, w_ref, o_ref):
        x = jnp.concatenate([h_ref[0, 0], h_ref[0, 1]], axis=1)
        y = jnp.maximum(jnp.dot(x, w_ref[0],
                                preferred_element_type=jnp.float32), 0.0)
        o_ref[0, 0] = y[:, :H]
        o_ref[0, 1] = y[:, H:]

    return pl.pallas_call(
        body,
        grid=(4, NPU // _TCB),
        in_specs=[
            pl.BlockSpec((1, 2, _TCB, H), lambda a, i: (a, 0, i, 0)),
            pl.BlockSpec((1, D, D), lambda a, i: (a, 0, 0)),
        ],
        out_specs=pl.BlockSpec((1, 2, _TCB, H), lambda a, i: (a, 0, i, 0)),
        out_shape=jax.ShapeDtypeStruct((4, 2, NPU, H), jnp.float32),
    )(h1, W1s)


def _tc_out(h2, W2s):
    def body(h_ref, w_ref, o_ref):
        x = jnp.concatenate([h_ref[0, 0], h_ref[0, 1]], axis=1)
        o_ref[...] = jnp.dot(x, w_ref[0],
                             preferred_element_type=jnp.float32)[None]

    return pl.pallas_call(
        body,
        grid=(4, NPU // _TCB),
        in_specs=[
            pl.BlockSpec((1, 2, _TCB, H), lambda a, i: (a, 0, i, 0)),
            pl.BlockSpec((1, D, D), lambda a, i: (a, 0, 0)),
        ],
        out_specs=pl.BlockSpec((1, _TCB, D), lambda a, i: (a, i, 0)),
        out_shape=jax.ShapeDtypeStruct((4, NPU, D), jnp.float32),
    )(h2, W2s)


def _tc_info_nce(a1, b1, a2, b2, alpha):
    RB = 512

    def body(a1_ref, b1_ref, a2_ref, b2_ref, o_ref, z1s, z2s, accs):
        z1 = alpha * a1_ref[...] + b1_ref[...]
        z2 = alpha * a2_ref[...] + b2_ref[...]
        z1 = z1 / (jnp.sqrt(jnp.sum(z1 * z1, axis=1, keepdims=True)) + 1e-8)
        z2 = z2 / (jnp.sqrt(jnp.sum(z2 * z2, axis=1, keepdims=True)) + 1e-8)
        z1s[...] = z1
        z2s[...] = z2

        def blk(i, acc):
            zb = z1s[pl.ds(i * RB, RB), :]
            sc = lax.dot_general(zb, z2s[...],
                                 (((1,), (1,)), ((), ())),
                                 preferred_element_type=jnp.float32) / TEMP
            m = jnp.max(sc, axis=1, keepdims=True)
            lse = jnp.log(jnp.sum(jnp.exp(sc - m), axis=1)) + m[:, 0]
            pos = jnp.sum(zb * z2s[pl.ds(i * RB, RB), :], axis=1) / TEMP
            return acc + jnp.sum(lse - pos)

        tot = lax.fori_loop(0, B // RB, blk, jnp.float32(0.0))
        o_ref[...] = jnp.full((8, 128), tot / B, jnp.float32)

    out = pl.pallas_call(
        body,
        out_shape=jax.ShapeDtypeStruct((8, 128), jnp.float32),
        scratch_shapes=[
            pltpu.VMEM((B, D), jnp.float32),
            pltpu.VMEM((B, D), jnp.float32),
            pltpu.VMEM((1, 1), jnp.float32),
        ],
    )(a1, b1, a2, b2)
    return out[0, 0]


def _tc_bpr_reg(ueS, peS, neS, ge):
    def body(u_ref, p_ref, n_ref, e_ref, o_ref):
        ue = u_ref[...] / 3.0
        pe = p_ref[...] / 3.0
        ne = n_ref[...] / 3.0
        d = jnp.sum(ue * pe, axis=1) - jnp.sum(ue * ne, axis=1)
        bpr = jnp.mean(jnp.where(
            d > 0,
            -jnp.log1p(jnp.exp(-d)),
            d - jnp.log1p(jnp.exp(d))))
        reg = REG * 0.5 * jnp.sum(e_ref[...] ** 2) / B
        o_ref[...] = jnp.concatenate(
            [jnp.full((1, 128), -bpr, jnp.float32),
             jnp.full((1, 128), reg, jnp.float32),
             jnp.zeros((6, 128), jnp.float32)], axis=0)

    out = pl.pallas_call(
        body,
        out_shape=jax.ShapeDtypeStruct((8, 128), jnp.float32),
    )(ueS, peS, neS, ge)
    return out[:2, 0]


def _tc_kl(hu0, hu1, hi0, hi1, Wuc, buc, Wic, bic):
    def body(u0, u1, i0, i1, wu, bu, wi, bi, o_ref):
        mask = (lax.broadcasted_iota(jnp.int32, (NPU, 1), 0) < NU)

        def klterm(h0, h1, w, b, nn_):
            node = jnp.dot(0.5 * (h0[...] + h1[...]), w[...],
                           preferred_element_type=jnp.float32) + b[0]
            mu = node[:, :IB2]
            st = jnp.logaddexp(node[:, IB2:] - IB2, 0.0)
            st = jnp.where(mask, st, 1.0)
            mu = jnp.where(mask, mu, 0.0)
            t = 1.0 + 2.0 * jnp.log(st) - mu * mu - st * st
            return -0.5 * (jnp.sum(t) / nn_) / jnp.log(2.0)

        ukl = klterm(u0, u1, wu, bu, NU)
        ikl = klterm(i0, i1, wi, bi, NI)
        o_ref[...] = jnp.concatenate(
            [jnp.full((1, 128), ukl, jnp.float32),
             jnp.full((1, 128), ikl, jnp.float32),
             jnp.zeros((6, 128), jnp.float32)], axis=0)

    out = pl.pallas_call(
        body,
        out_shape=jax.ShapeDtypeStruct((8, 128), jnp.float32),
    )(hu0, hu1, hi0, hi1, Wuc, buc[None, :], Wic, bic[None, :])
    return out[:2, 0]


def _info_nce(z1, z2, t):
    z1 = z1 / (jnp.linalg.norm(z1, axis=1, keepdims=True) + 1e-8)
    z2 = z2 / (jnp.linalg.norm(z2, axis=1, keepdims=True) + 1e-8)
    pos = jnp.sum(z1 * z2, axis=1) / t
    ttl = (z1 @ z2.T) / t
    return jnp.mean(jax.scipy.special.logsumexp(ttl, axis=1) - pos)


def _stack5_kernel(x_ref, o_ref):
    o_ref[...] = x_ref[...]


def _stack5(vals):
    x = jnp.broadcast_to(jnp.pad(jnp.stack(vals), (0, 3))[:, None], (8, 128))
    out = pl.pallas_call(
        _stack5_kernel,
        out_shape=jax.ShapeDtypeStruct((8, 128), jnp.float32),
    )(x)
    return out[:5, 0]


def kernel(user_table, item_table, W_vl, Wu0a, Wu0b, Wu1a, Wu1b, Wi0a, Wi0b,
           Wi1a, Wi1b, Wuc, buc, Wic, bic, adj_src, adj_dst, uu0_src, uu0_dst,
           uu1_src, uu1_dst, ii0_src, ii0_dst, ii1_src, ii1_dst, user,
           positive, negative):
    npad_e = EP - E
    adj2 = jnp.stack([
        jnp.concatenate([adj_src, jnp.full((npad_e,), N, jnp.int32)]),
        jnp.concatenate([adj_dst, jnp.full((npad_e,), N, jnp.int32)]),
    ])
    rsrd = _sc_degree(adj2)
    rs = rsrd[0]

    x0 = jnp.concatenate([user_table, item_table], axis=0)
    x0p = jnp.pad(x0, ((0, NP - N), (0, 0)))
    xs0_flat = _tc_prescale(x0p, rsrd)
    zrows = jnp.zeros((EC, H), jnp.float32)
    S2 = _sc_gcn3(xs0_flat, adj2, rsrd, zrows)

    up = jnp.pad(user_table, ((0, NPU - NU), (0, 0)))
    itp = jnp.pad(item_table, ((0, NPU - NI), (0, 0)))
    tabs = jnp.stack([up, itp])
    hvl_flat = _tc_hvl(tabs, W_vl)

    npad_h = EHP - EH

    def pads(a):
        return jnp.concatenate([a, jnp.full((npad_h,), NU, jnp.int32)])

    edges4 = jnp.stack([
        jnp.stack([pads(uu0_src), pads(uu0_dst)]),
        jnp.stack([pads(uu1_src), pads(uu1_dst)]),
        jnp.stack([pads(ii0_src), pads(ii0_dst)]),
        jnp.stack([pads(ii1_src), pads(ii1_dst)]),
    ])
    ew4 = _sc_dots(hvl_flat, edges4)

    emb_flat = jnp.concatenate(
        [up[:, :H], up[:, H:], itp[:, :H], itp[:, H:]], axis=0)
    h1 = _sc_seg(emb_flat, edges4, ew4, zrows, (0, 0, 1, 1))
    W1s = jnp.stack([Wu0a, Wu1a, Wi0a, Wi1a])
    hr = _tc_mid(h1, W1s)
    h2 = _sc_seg(hr.reshape(8 * NPU, H), edges4, ew4, zrows, (0, 1, 2, 3))
    W2s = jnp.stack([Wu0b, Wu1b, Wi0b, Wi1b])
    hh = _tc_out(h2, W2s)

    klv = _tc_kl(hh[0], hh[1], hh[2], hh[3], Wuc, buc, Wic, bic)
    ib_loss = IBL * (klv[0] + klv[1])

    oS, oh, oe = _sc_bgather(S2.reshape(2 * NP, H), hh.reshape(4 * NPU, D),
                             jnp.concatenate([up, itp], axis=0),
                             user, positive, negative)
    ueS = jnp.concatenate([oS[0, 0], oS[1, 0]], axis=1)
    peS = jnp.concatenate([oS[0, 1], oS[1, 1]], axis=1)
    neS = jnp.concatenate([oS[0, 2], oS[1, 2]], axis=1)
    br = _tc_bpr_reg(ueS, peS, neS, oe.reshape(3 * B, D))
    bpr_loss, reg_loss = br[0], br[1]

    third = 1.0 / 3.0
    ssl_loss = SSL * (_tc_info_nce(ueS, oh[0], ueS, oh[1], third)
                      + _tc_info_nce(peS, oh[2], peS, oh[3], third))
    intra_loss = INTRA * (_tc_info_nce(oh[0], oh[0], oh[1], oh[1], 0.0)
                          + _tc_info_nce(oh[2], oh[2], oh[3], oh[3], 0.0))
    return _stack5([bpr_loss, reg_loss, ssl_loss, intra_loss, ib_loss])


# speedup vs baseline: 3.1054x; 1.2969x over previous
import functools

import jax
import jax.numpy as jnp
from jax import lax
from jax.experimental import pallas as pl
from jax.experimental.pallas import tpu as pltpu
from jax.experimental.pallas import tpu_sc as plsc

NU = 10000
NI = 10000
N = NU + NI
D = 128
H = 64
B = 4096
E = 320000
EH = 160000
IB = 64
IB2 = 32
GCN_LAYERS = 3
TEMP = 0.2
REG = 1e-4
SSL = 0.1
IBL = 0.01
INTRA = 0.1


NC = 2
NS = 16
NP = 20480
NPU = 10240
EP = 327680
EHP = 163840
CE = 512
NCH_E = EP // NS // CE
NCH_H = EHP // NS // CE
RT = NP // NS
RTU = NPU // NS
EC = 64
CEG = 256
NCHG = EP // NS // CEG


def _sc_mesh():
    return plsc.VectorSubcoreMesh(core_axis_name="c", subcore_axis_name="s")


def _sc_params():
    return pltpu.CompilerParams(needs_layout_passes=False,
                                use_tc_tiling_on_sc=False)


def _zero_1d(ref, nwords):
    def z(i, _):
        ref[pl.ds(i * 16, 16)] = jnp.zeros((16,), jnp.float32)
        return _
    lax.fori_loop(0, nwords // 16, z, None)


def _zero_2d(ref, nrows):
    def z(i, _):
        for j in range(H // 16):
            ref[i, pl.ds(j * 16, 16)] = jnp.zeros((16,), jnp.float32)
        return _
    lax.fori_loop(0, nrows, z, None)


def _rsqrt16(d):
    i = plsc.bitcast(d, jnp.int32)
    y = plsc.bitcast(jnp.int32(0x5F3759DF) - (i >> 1), jnp.float32)
    for _ in range(3):
        y = y * (1.5 - 0.5 * d * y * y)
    return y


def _sc_degree(adj2):
    @functools.partial(
        pl.kernel, mesh=_sc_mesh(), compiler_params=_sc_params(),
        out_type=jax.ShapeDtypeStruct((2, NP), jnp.float32),
        scratch_types=[
            pltpu.VMEM((NP,), jnp.float32),
            pltpu.VMEM((CE,), jnp.int32),
            pltpu.VMEM_SHARED((NS, NP), jnp.float32),
            pltpu.VMEM((RT,), jnp.float32),
            pltpu.VMEM((RT,), jnp.float32),
        ],
    )
    def k(adj, rsrd, hist, idxb, shared, tmp, red):
        c = lax.axis_index("c")
        s = lax.axis_index("s")
        _zero_1d(hist, NP)
        ones = jnp.ones((16,), jnp.float32)

        def chunk(ch, _):
            off = (s * NCH_E + ch) * CE
            pltpu.sync_copy(adj.at[c, pl.ds(off, CE)], idxb)

            def vec(j, x):
                v = idxb[pl.ds(j * 16, 16)]
                plsc.addupdate_scatter(hist, [v], ones)
                return x
            lax.fori_loop(0, CE // 16, vec, None)
            return _
        lax.fori_loop(0, NCH_E, chunk, None)
        pltpu.sync_copy(hist, shared.at[s])
        plsc.subcore_barrier()

        _zero_1d(red, RT)
        for t in range(NS):
            pltpu.sync_copy(shared.at[t, pl.ds(s * RT, RT)], tmp)

            def acc(j, _):
                red[pl.ds(j * 16, 16)] = (red[pl.ds(j * 16, 16)]
                                          + tmp[pl.ds(j * 16, 16)])
                return _
            lax.fori_loop(0, RT // 16, acc, None)

        def rq(j, _):
            d = jnp.maximum(red[pl.ds(j * 16, 16)], 1.0)
            red[pl.ds(j * 16, 16)] = _rsqrt16(d)
            return _
        lax.fori_loop(0, RT // 16, rq, None)
        pltpu.sync_copy(red, rsrd.at[c, pl.ds(s * RT, RT)])

    return k(adj2)


def _sc_gcn3(xs0_flat, adj2, rsrd, zrows):
    outs = (jax.ShapeDtypeStruct((2, NP, H), jnp.float32),
            jax.ShapeDtypeStruct((2 * NP, H), jnp.float32))

    CG = 128
    NCH = EP // NS // CG
    ECg = 128
    NEC = RT // ECg

    @functools.partial(
        pl.kernel, mesh=_sc_mesh(), compiler_params=_sc_params(),
        out_type=outs,
        scratch_types=[
            pltpu.VMEM_SHARED((NP, H), jnp.float32),
            pltpu.VMEM((2, CG), jnp.int32),
            pltpu.VMEM((2, CG), jnp.int32),
            pltpu.VMEM((2, CG, H), jnp.float32),
            pltpu.VMEM((ECg, H), jnp.float32),
            pltpu.VMEM((ECg, H), jnp.float32),
            pltpu.VMEM((ECg, H), jnp.float32),
            pltpu.VMEM((RT,), jnp.float32),
            pltpu.VMEM((RT,), jnp.float32),
            pltpu.SemaphoreType.DMA((2,)),
            pltpu.SemaphoreType.DMA((2,)),
        ],
    )
    def k(xs0, adj, rr, zr, S, xsw, accum, srcb, dstb, rows, abuf,
          sbuf, zbuf, rsb, rdb, gsem, ssem):
        c = lax.axis_index("c")
        s = lax.axis_index("s")
        pltpu.sync_copy(rr.at[0, pl.ds(s * RT, RT)], rsb)
        pltpu.sync_copy(rr.at[1, pl.ds(s * RT, RT)], rdb)
        _zero_2d(zbuf, ECg)
        for e in range(NEC):
            pltpu.sync_copy(zbuf, accum.at[pl.ds(s * RT + e * ECg, ECg)])
        plsc.subcore_barrier()

        coff = c * NP

        for l in range(GCN_LAYERS):
            tbl = xs0 if l == 0 else xsw

            def issue_gather(ch, bb):
                off = (s * NCH + ch) * CG
                pltpu.sync_copy(adj.at[0, pl.ds(off, CG)], srcb.at[bb])
                pltpu.sync_copy(adj.at[1, pl.ds(off, CG)], dstb.at[bb])

                def mk(j, x):
                    srcb[bb, pl.ds(j * 16, 16)] = (
                        srcb[bb, pl.ds(j * 16, 16)] + coff)
                    return x
                lax.fori_loop(0, CG // 16, mk, None)
                pltpu.async_copy(tbl.at[srcb.at[bb]], rows.at[bb],
                                 gsem.at[bb])

            def wait_gather(bb):
                pltpu.make_async_copy(tbl.at[srcb.at[bb]], rows.at[bb],
                                      gsem.at[bb]).wait()

            def issue_scatter(bb):
                pltpu.async_copy(rows.at[bb], accum.at[dstb.at[bb]],
                                 ssem.at[bb], add=True)

            def wait_scatter(bb):
                pltpu.make_async_copy(rows.at[bb], accum.at[dstb.at[bb]],
                                      ssem.at[bb]).wait()

            issue_gather(0, 0)

            def outer(o, _):
                ch0 = 2 * o

                @pl.when(o > 0)
                def _w():
                    wait_scatter(1)
                issue_gather(ch0 + 1, 1)
                wait_gather(0)
                issue_scatter(0)

                @pl.when(o < NCH // 2 - 1)
                def _n():
                    wait_scatter(0)
                    issue_gather(ch0 + 2, 0)
                wait_gather(1)
                issue_scatter(1)
                return _
            lax.fori_loop(0, NCH // 2, outer, None)
            wait_scatter(0)
            wait_scatter(1)
            plsc.subcore_barrier()

            for e in range(NEC):
                r0 = s * RT + e * ECg
                pltpu.sync_copy(accum.at[pl.ds(r0, ECg)], abuf)
                pltpu.sync_copy(zbuf, accum.at[pl.ds(r0, ECg)])
                if l > 0:
                    pltpu.sync_copy(S.at[c, pl.ds(r0, ECg)], sbuf)

                def rowfn(i, _):
                    rix = jnp.full((16,), e * ECg + i, jnp.int32)
                    rdv = plsc.load_gather(rdb, [rix])
                    for j in range(H // 16):
                        xv = abuf[i, pl.ds(j * 16, 16)] * rdv
                        abuf[i, pl.ds(j * 16, 16)] = xv
                        if l > 0:
                            sbuf[i, pl.ds(j * 16, 16)] = (
                                sbuf[i, pl.ds(j * 16, 16)] + xv)
                        else:
                            sbuf[i, pl.ds(j * 16, 16)] = xv
                    return _
                lax.fori_loop(0, ECg, rowfn, None)
                pltpu.sync_copy(sbuf, S.at[c, pl.ds(r0, ECg)])
                if l < GCN_LAYERS - 1:
                    def rowfn2(i, _):
                        rix = jnp.full((16,), e * ECg + i, jnp.int32)
                        rsv = plsc.load_gather(rsb, [rix])
                        for j in range(H // 16):
                            sbuf[i, pl.ds(j * 16, 16)] = (
                                abuf[i, pl.ds(j * 16, 16)] * rsv)
                        return _
                    lax.fori_loop(0, ECg, rowfn2, None)
                    pltpu.sync_copy(sbuf, xsw.at[pl.ds(coff + r0, ECg)])
            plsc.subcore_barrier()

    return k(xs0_flat, adj2, rsrd, zrows)[0]


def _sc_dots(hvl_flat, edges4):
    CD = 128
    NCHD = EHP // NS // CD

    @functools.partial(
        pl.kernel, mesh=_sc_mesh(), compiler_params=_sc_params(),
        out_type=jax.ShapeDtypeStruct((4, EHP), jnp.float32),
        scratch_types=[
            pltpu.VMEM((2, CD), jnp.int32),
            pltpu.VMEM((2, CD), jnp.int32),
            pltpu.VMEM((2, CD, D), jnp.float32),
            pltpu.VMEM((2, CD, D), jnp.float32),
            pltpu.VMEM((CD,), jnp.float32),
            pltpu.SemaphoreType.DMA((2,)),
            pltpu.SemaphoreType.DMA((2,)),
        ],
    )
    def k(hvl, edges, ew, srcb, dstb, hs, hd, ewb, sema, semb):
        c = lax.axis_index("c")
        s = lax.axis_index("s")
        coff = c * NPU
        lane0 = lax.iota(jnp.int32, 16) == 0
        for kk in range(2):
            si = 2 * c + kk

            def issue(ch, bb):
                off = (s * NCHD + ch) * CD
                pltpu.sync_copy(edges.at[si, 0, pl.ds(off, CD)], srcb.at[bb])
                pltpu.sync_copy(edges.at[si, 1, pl.ds(off, CD)], dstb.at[bb])

                def mk(j, x):
                    srcb[bb, pl.ds(j * 16, 16)] = (
                        srcb[bb, pl.ds(j * 16, 16)] + coff)
                    dstb[bb, pl.ds(j * 16, 16)] = (
                        dstb[bb, pl.ds(j * 16, 16)] + coff)
                    return x
                lax.fori_loop(0, CD // 16, mk, None)
                pltpu.async_copy(hvl.at[srcb.at[bb]], hs.at[bb], sema.at[bb])
                pltpu.async_copy(hvl.at[dstb.at[bb]], hd.at[bb], semb.at[bb])

            def wait(bb):
                pltpu.make_async_copy(hvl.at[srcb.at[bb]], hs.at[bb],
                                      sema.at[bb]).wait()
                pltpu.make_async_copy(hvl.at[dstb.at[bb]], hd.at[bb],
                                      semb.at[bb]).wait()

            def compute(ch, bb):
                def dot1(i, x):
                    acc = hs[bb, i, pl.ds(0, 16)] * hd[bb, i, pl.ds(0, 16)]
                    for j in range(1, D // 16):
                        acc = acc + (hs[bb, i, pl.ds(j * 16, 16)]
                                     * hd[bb, i, pl.ds(j * 16, 16)])
                    dv = jnp.full((16,), jnp.sum(acc), jnp.float32)
                    plsc.store_scatter(ewb, [jnp.full((16,), i, jnp.int32)],
                                       dv, mask=lane0)
                    return x
                lax.fori_loop(0, CD, dot1, None)

                def sig(j, x):
                    v = ewb[pl.ds(j * 16, 16)]
                    ewb[pl.ds(j * 16, 16)] = 1.0 / (1.0 + jnp.exp(-v))
                    return x
                lax.fori_loop(0, CD // 16, sig, None)
                off = (s * NCHD + ch) * CD
                pltpu.sync_copy(ewb, ew.at[si, pl.ds(off, CD)])

            issue(0, 0)

            def outer(o, _):
                ch0 = 2 * o
                issue(ch0 + 1, 1)
                wait(0)
                compute(ch0, 0)

                @pl.when(o < NCHD // 2 - 1)
                def _n():
                    issue(ch0 + 2, 0)
                wait(1)
                compute(ch0 + 1, 1)
                return _
            lax.fori_loop(0, NCHD // 2, outer, None)

    return k(hvl_flat, edges4)


def _sc_seg(tbl_flat, edges4, ew4, zrows, tsel):
    CS = 512
    NCHS = EHP // NS // CS
    ECs = 128
    NECS = RTU // ECs

    @functools.partial(
        pl.kernel, mesh=_sc_mesh(), compiler_params=_sc_params(),
        out_type=jax.ShapeDtypeStruct((4, 2, NPU, H), jnp.float32),
        scratch_types=[
            pltpu.VMEM_SHARED((NPU, H), jnp.float32),
            pltpu.VMEM((2, CS), jnp.int32),
            pltpu.VMEM((2, CS), jnp.int32),
            pltpu.VMEM((2, CS), jnp.float32),
            pltpu.VMEM((2, CS, H), jnp.float32),
            pltpu.VMEM((ECs, H), jnp.float32),
            pltpu.SemaphoreType.DMA((2,)),
            pltpu.SemaphoreType.DMA((2,)),
        ],
    )
    def k(tbl, edges, ew, zr, out, accum, srcb, dstb, ewb, rows,
          zbuf, gsem, ssem):
        c = lax.axis_index("c")
        s = lax.axis_index("s")
        _zero_2d(zbuf, ECs)
        for ae in range(4):
            off0 = tsel[ae] * 2 * NPU
            for e in range(NECS):
                pltpu.sync_copy(zbuf, accum.at[pl.ds(s * RTU + e * ECs,
                                                     ECs)])
            plsc.subcore_barrier()

            def issue_gather(ch, bb):
                off = (s * NCHS + ch) * CS
                pltpu.sync_copy(edges.at[ae, 0, pl.ds(off, CS)], srcb.at[bb])
                pltpu.sync_copy(edges.at[ae, 1, pl.ds(off, CS)], dstb.at[bb])
                pltpu.sync_copy(ew.at[ae, pl.ds(off, CS)], ewb.at[bb])

                def mk(j, x):
                    srcb[bb, pl.ds(j * 16, 16)] = (
                        srcb[bb, pl.ds(j * 16, 16)] + (off0 + c * NPU))
                    return x
                lax.fori_loop(0, CS // 16, mk, None)
                pltpu.async_copy(tbl.at[srcb.at[bb]], rows.at[bb],
                                 gsem.at[bb])

            def wait_gather(bb):
                pltpu.make_async_copy(tbl.at[srcb.at[bb]], rows.at[bb],
                                      gsem.at[bb]).wait()

            def weight(bb):
                def wrow(i, y):
                    for u in range(2):
                        r = 2 * i + u
                        wv = plsc.load_gather(
                            ewb.at[bb], [jnp.full((16,), r, jnp.int32)])
                        for j in range(H // 16):
                            rows[bb, r, pl.ds(j * 16, 16)] = (
                                rows[bb, r, pl.ds(j * 16, 16)] * wv)
                    return y
                lax.fori_loop(0, CS // 2, wrow, None)

            def issue_scatter(bb):
                pltpu.async_copy(rows.at[bb], accum.at[dstb.at[bb]],
                                 ssem.at[bb], add=True)

            def wait_scatter(bb):
                pltpu.make_async_copy(rows.at[bb], accum.at[dstb.at[bb]],
                                      ssem.at[bb]).wait()

            issue_gather(0, 0)

            def outer(o, _):
                ch0 = 2 * o

                @pl.when(o > 0)
                def _w():
                    wait_scatter(1)
                issue_gather(ch0 + 1, 1)
                wait_gather(0)
                weight(0)
                issue_scatter(0)

                @pl.when(o < NCHS // 2 - 1)
                def _n():
                    wait_scatter(0)
                    issue_gather(ch0 + 2, 0)
                wait_gather(1)
                weight(1)
                issue_scatter(1)
                return _
            lax.fori_loop(0, NCHS // 2, outer, None)
            wait_scatter(0)
            wait_scatter(1)
            plsc.subcore_barrier()
            for e in range(NECS):
                r0 = s * RTU + e * ECs
                pltpu.sync_copy(accum.at[pl.ds(r0, ECs)],
                                out.at[ae, c, pl.ds(r0, ECs)])

    return k(tbl_flat, edges4, ew4, zrows)


def _sc_bgather(S_flat, h_flat, ego_flat, user, positive, negative):
    BW = B // (NC * NS)

    outs = (jax.ShapeDtypeStruct((2, 3, B, H), jnp.float32),
            jax.ShapeDtypeStruct((4, B, D), jnp.float32),
            jax.ShapeDtypeStruct((3, B, D), jnp.float32))

    @functools.partial(
        pl.kernel, mesh=_sc_mesh(), compiler_params=_sc_params(),
        out_type=outs,
        scratch_types=[
            pltpu.VMEM((BW,), jnp.int32),
            pltpu.VMEM((BW,), jnp.int32),
            pltpu.VMEM((BW,), jnp.int32),
            pltpu.VMEM((BW,), jnp.int32),
            pltpu.VMEM((BW, H), jnp.float32),
            pltpu.VMEM((BW, D), jnp.float32),
            pltpu.SemaphoreType.DMA,
        ],
    )
    def k(Sf, hf, ef, iu, ip, inn, oS, oh, oe, idxu, idxp, idxn, idxg,
          rows64, rows128, sem):
        c = lax.axis_index("c")
        s = lax.axis_index("s")
        w = s * NC + c
        b0 = w * BW
        pltpu.sync_copy(iu.at[pl.ds(b0, BW)], idxu)
        pltpu.sync_copy(ip.at[pl.ds(b0, BW)], idxp)
        pltpu.sync_copy(inn.at[pl.ds(b0, BW)], idxn)

        def mkidx(src_ref, off):
            def go(j, x):
                idxg[pl.ds(j * 16, 16)] = src_ref[pl.ds(j * 16, 16)] + off
                return x
            lax.fori_loop(0, BW // 16, go, None)

        for half in range(2):
            for slot, (iref, noff) in enumerate(
                    [(idxu, 0), (idxp, NU), (idxn, NU)]):
                mkidx(iref, half * NP + noff)
                pltpu.async_copy(Sf.at[idxg], rows64, sem).wait()
                pltpu.sync_copy(rows64, oS.at[half, slot, pl.ds(b0, BW)])
        for ae, iref in enumerate([idxu, idxu, idxp, idxp]):
            mkidx(iref, ae * NPU)
            pltpu.async_copy(hf.at[idxg], rows128, sem).wait()
            pltpu.sync_copy(rows128, oh.at[ae, pl.ds(b0, BW)])
        for slot, (iref, noff) in enumerate(
                [(idxu, 0), (idxp, NPU), (idxn, NPU)]):
            mkidx(iref, noff)
            pltpu.async_copy(ef.at[idxg], rows128, sem).wait()
            pltpu.sync_copy(rows128, oe.at[slot, pl.ds(b0, BW)])

    return k(S_flat, h_flat, ego_flat, user, positive, negative)


_TCB = 512


def _tc_prescale(x0p, rsrd):
    def body(x_ref, r_ref, o_ref):
        rs = r_ref[...][0][:, None]
        xs = x_ref[...] * rs
        o_ref[0] = xs[:, :H]
        o_ref[1] = xs[:, H:]

    out = pl.pallas_call(
        body,
        grid=(NP // _TCB,),
        in_specs=[
            pl.BlockSpec((_TCB, D), lambda i: (i, 0)),
            pl.BlockSpec((2, _TCB), lambda i: (0, i)),
        ],
        out_specs=pl.BlockSpec((2, _TCB, H), lambda i: (0, i, 0)),
        out_shape=jax.ShapeDtypeStruct((2, NP, H), jnp.float32),
    )(x0p, rsrd)
    return out.reshape(2 * NP, H)


def _tc_hvl(tabs, W_vl):
    def body(x_ref, w_ref, o_ref):
        o_ref[...] = jnp.dot(x_ref[0], w_ref[...],
                             preferred_element_type=jnp.float32)[None]

    out = pl.pallas_call(
        body,
        grid=(2, NPU // _TCB),
        in_specs=[
            pl.BlockSpec((1, _TCB, D), lambda t, i: (t, i, 0)),
            pl.BlockSpec((D, D), lambda t, i: (0, 0)),
        ],
        out_specs=pl.BlockSpec((1, _TCB, D), lambda t, i: (t, i, 0)),
        out_shape=jax.ShapeDtypeStruct((2, NPU, D), jnp.float32),
    )(tabs, W_vl)
    return out.reshape(2 * NPU, D)


def _tc_mid(h1, W1s):
    def body(h_ref, w_ref, o_ref):
        x = jnp.concatenate([h_ref[0, 0], h_ref[0, 1]], axis=1)
        y = jnp.maximum(jnp.dot(x, w_ref[0],
                                preferred_element_type=jnp.float32), 0.0)
        o_ref[0, 0] = y[:, :H]
        o_ref[0, 1] = y[:, H:]

    return pl.pallas_call(
        body,
        grid=(4, NPU // _TCB),
        in_specs=[
            pl.BlockSpec((1, 2, _TCB, H), lambda a, i: (a, 0, i, 0)),
            pl.BlockSpec((1, D, D), lambda a, i: (a, 0, 0)),
        ],
        out_specs=pl.BlockSpec((1, 2, _TCB, H), lambda a, i: (a, 0, i, 0)),
        out_shape=jax.ShapeDtypeStruct((4, 2, NPU, H), jnp.float32),
    )(h1, W1s)


def _tc_out(h2, W2s):
    def body(h_ref, w_ref, o_ref):
        x = jnp.concatenate([h_ref[0, 0], h_ref[0, 1]], axis=1)
        o_ref[...] = jnp.dot(x, w_ref[0],
                             preferred_element_type=jnp.float32)[None]

    return pl.pallas_call(
        body,
        grid=(4, NPU // _TCB),
        in_specs=[
            pl.BlockSpec((1, 2, _TCB, H), lambda a, i: (a, 0, i, 0)),
            pl.BlockSpec((1, D, D), lambda a, i: (a, 0, 0)),
        ],
        out_specs=pl.BlockSpec((1, _TCB, D), lambda a, i: (a, i, 0)),
        out_shape=jax.ShapeDtypeStruct((4, NPU, D), jnp.float32),
    )(h2, W2s)


def _tc_info_nce(a1, b1, a2, b2, alpha):
    RB = 512

    def body(a1_ref, b1_ref, a2_ref, b2_ref, o_ref, z1s, z2s, accs):
        z1 = alpha * a1_ref[...] + b1_ref[...]
        z2 = alpha * a2_ref[...] + b2_ref[...]
        z1 = z1 / (jnp.sqrt(jnp.sum(z1 * z1, axis=1, keepdims=True)) + 1e-8)
        z2 = z2 / (jnp.sqrt(jnp.sum(z2 * z2, axis=1, keepdims=True)) + 1e-8)
        z1s[...] = z1
        z2s[...] = z2

        def blk(i, acc):
            zb = z1s[pl.ds(i * RB, RB), :]
            sc = lax.dot_general(zb, z2s[...],
                                 (((1,), (1,)), ((), ())),
                                 preferred_element_type=jnp.float32) / TEMP
            m = jnp.max(sc, axis=1, keepdims=True)
            lse = jnp.log(jnp.sum(jnp.exp(sc - m), axis=1)) + m[:, 0]
            pos = jnp.sum(zb * z2s[pl.ds(i * RB, RB), :], axis=1) / TEMP
            return acc + jnp.sum(lse - pos)

        tot = lax.fori_loop(0, B // RB, blk, jnp.float32(0.0))
        o_ref[...] = jnp.full((8, 128), tot / B, jnp.float32)

    out = pl.pallas_call(
        body,
        out_shape=jax.ShapeDtypeStruct((8, 128), jnp.float32),
        scratch_shapes=[
            pltpu.VMEM((B, D), jnp.float32),
            pltpu.VMEM((B, D), jnp.float32),
            pltpu.VMEM((1, 1), jnp.float32),
        ],
    )(a1, b1, a2, b2)
    return out[0, 0]


def _tc_bpr_reg(ueS, peS, neS, ge):
    def body(u_ref, p_ref, n_ref, e_ref, o_ref):
        ue = u_ref[...] / 3.0
        pe = p_ref[...] / 3.0
        ne = n_ref[...] / 3.0
        d = jnp.sum(ue * pe, axis=1) - jnp.sum(ue * ne, axis=1)
        bpr = jnp.mean(jnp.where(
            d > 0,
            -jnp.log1p(jnp.exp(-d)),
            d - jnp.log1p(jnp.exp(d))))
        reg = REG * 0.5 * jnp.sum(e_ref[...] ** 2) / B
        o_ref[...] = jnp.concatenate(
            [jnp.full((1, 128), -bpr, jnp.float32),
             jnp.full((1, 128), reg, jnp.float32),
             jnp.zeros((6, 128), jnp.float32)], axis=0)

    out = pl.pallas_call(
        body,
        out_shape=jax.ShapeDtypeStruct((8, 128), jnp.float32),
    )(ueS, peS, neS, ge)
    return out[:2, 0]


def _tc_kl(hu0, hu1, hi0, hi1, Wuc, buc, Wic, bic):
    def body(u0, u1, i0, i1, wu, bu, wi, bi, o_ref):
        mask = (lax.broadcasted_iota(jnp.int32, (NPU, 1), 0) < NU)

        def klterm(h0, h1, w, b, nn_):
            node = jnp.dot(0.5 * (h0[...] + h1[...]), w[...],
                           preferred_element_type=jnp.float32) + b[0]
            mu = node[:, :IB2]
            st = jnp.logaddexp(node[:, IB2:] - IB2, 0.0)
            st = jnp.where(mask, st, 1.0)
            mu = jnp.where(mask, mu, 0.0)
            t = 1.0 + 2.0 * jnp.log(st) - mu * mu - st * st
            return -0.5 * (jnp.sum(t) / nn_) / jnp.log(2.0)

        ukl = klterm(u0, u1, wu, bu, NU)
        ikl = klterm(i0, i1, wi, bi, NI)
        o_ref[...] = jnp.concatenate(
            [jnp.full((1, 128), ukl, jnp.float32),
             jnp.full((1, 128), ikl, jnp.float32),
             jnp.zeros((6, 128), jnp.float32)], axis=0)

    out = pl.pallas_call(
        body,
        out_shape=jax.ShapeDtypeStruct((8, 128), jnp.float32),
    )(hu0, hu1, hi0, hi1, Wuc, buc[None, :], Wic, bic[None, :])
    return out[:2, 0]


def _info_nce(z1, z2, t):
    z1 = z1 / (jnp.linalg.norm(z1, axis=1, keepdims=True) + 1e-8)
    z2 = z2 / (jnp.linalg.norm(z2, axis=1, keepdims=True) + 1e-8)
    pos = jnp.sum(z1 * z2, axis=1) / t
    ttl = (z1 @ z2.T) / t
    return jnp.mean(jax.scipy.special.logsumexp(ttl, axis=1) - pos)


def _stack5_kernel(x_ref, o_ref):
    o_ref[...] = x_ref[...]


def _stack5(vals):
    x = jnp.broadcast_to(jnp.pad(jnp.stack(vals), (0, 3))[:, None], (8, 128))
    out = pl.pallas_call(
        _stack5_kernel,
        out_shape=jax.ShapeDtypeStruct((8, 128), jnp.float32),
    )(x)
    return out[:5, 0]


def kernel(user_table, item_table, W_vl, Wu0a, Wu0b, Wu1a, Wu1b, Wi0a, Wi0b,
           Wi1a, Wi1b, Wuc, buc, Wic, bic, adj_src, adj_dst, uu0_src, uu0_dst,
           uu1_src, uu1_dst, ii0_src, ii0_dst, ii1_src, ii1_dst, user,
           positive, negative):
    npad_e = EP - E
    adj2 = jnp.stack([
        jnp.concatenate([adj_src, jnp.full((npad_e,), N, jnp.int32)]),
        jnp.concatenate([adj_dst, jnp.full((npad_e,), N, jnp.int32)]),
    ])
    rsrd = _sc_degree(adj2)
    rs = rsrd[0]

    x0 = jnp.concatenate([user_table, item_table], axis=0)
    x0p = jnp.pad(x0, ((0, NP - N), (0, 0)))
    xs0_flat = _tc_prescale(x0p, rsrd)
    zrows = jnp.zeros((EC, H), jnp.float32)
    S2 = _sc_gcn3(xs0_flat, adj2, rsrd, zrows)

    up = jnp.pad(user_table, ((0, NPU - NU), (0, 0)))
    itp = jnp.pad(item_table, ((0, NPU - NI), (0, 0)))
    tabs = jnp.stack([up, itp])
    hvl_flat = _tc_hvl(tabs, W_vl)

    npad_h = EHP - EH

    def pads(a):
        return jnp.concatenate([a, jnp.full((npad_h,), NU, jnp.int32)])

    edges4 = jnp.stack([
        jnp.stack([pads(uu0_src), pads(uu0_dst)]),
        jnp.stack([pads(uu1_src), pads(uu1_dst)]),
        jnp.stack([pads(ii0_src), pads(ii0_dst)]),
        jnp.stack([pads(ii1_src), pads(ii1_dst)]),
    ])
    ew4 = _sc_dots(hvl_flat, edges4)

    emb_flat = jnp.concatenate(
        [up[:, :H], up[:, H:], itp[:, :H], itp[:, H:]], axis=0)
    h1 = _sc_seg(emb_flat, edges4, ew4, zrows, (0, 0, 1, 1))
    W1s = jnp.stack([Wu0a, Wu1a, Wi0a, Wi1a])
    hr = _tc_mid(h1, W1s)
    h2 = _sc_seg(hr.reshape(8 * NPU, H), edges4, ew4, zrows, (0, 1, 2, 3))
    W2s = jnp.stack([Wu0b, Wu1b, Wi0b, Wi1b])
    hh = _tc_out(h2, W2s)

    klv = _tc_kl(hh[0], hh[1], hh[2], hh[3], Wuc, buc, Wic, bic)
    ib_loss = IBL * (klv[0] + klv[1])

    oS, oh, oe = _sc_bgather(S2.reshape(2 * NP, H), hh.reshape(4 * NPU, D),
                             jnp.concatenate([up, itp], axis=0),
                             user, positive, negative)
    ueS = jnp.concatenate([oS[0, 0], oS[1, 0]], axis=1)
    peS = jnp.concatenate([oS[0, 1], oS[1, 1]], axis=1)
    neS = jnp.concatenate([oS[0, 2], oS[1, 2]], axis=1)
    br = _tc_bpr_reg(ueS, peS, neS, oe.reshape(3 * B, D))
    bpr_loss, reg_loss = br[0], br[1]

    third = 1.0 / 3.0
    ssl_loss = SSL * (_tc_info_nce(ueS, oh[0], ueS, oh[1], third)
                      + _tc_info_nce(peS, oh[2], peS, oh[3], third))
    intra_loss = INTRA * (_tc_info_nce(oh[0], oh[0], oh[1], oh[1], 0.0)
                          + _tc_info_nce(oh[2], oh[2], oh[3], oh[3], 0.0))
    return _stack5([bpr_loss, reg_loss, ssl_loss, intra_loss, ib_loss])
